# Initial kernel scaffold; baseline (speedup 1.0000x reference)
import jax
import jax.numpy as jnp
from jax.experimental import pallas as pl


def kernel(x, edge_index, W_gat, att_src, att_dst, b_gat, W_comp, b_comp, W_ih, W_hh, b_ih, b_hh, W_opt, b_opt):
    raise NotImplementedError("write your pallas kernel here")


# speedup vs baseline: 5.9877x; 5.9877x over previous
import functools

import jax
import jax.numpy as jnp
from jax import lax
from jax.experimental import pallas as pl
from jax.experimental.pallas import tpu as pltpu

_N = 10000
_D = 128
_H = 8
_BLK = 400


def _proj_body(x_ref, wt_ref, m_ref, xp_ref, a_ref):
    x = x_ref[...]
    xp = lax.dot_general(x, wt_ref[...], (((1,), (1,)), ((), ())),
                         preferred_element_type=jnp.float32)
    xp_ref[...] = xp
    a_ref[...] = jnp.dot(xp, m_ref[...], preferred_element_type=jnp.float32)


def _proj(x, W_gat, M):
    grid = (_N // _BLK,)
    return pl.pallas_call(
        _proj_body,
        grid=grid,
        in_specs=[
            pl.BlockSpec((_BLK, _D), lambda i: (i, 0)),
            pl.BlockSpec((_H * _D, _D), lambda i: (0, 0)),
            pl.BlockSpec((_H * _D, 2 * _H), lambda i: (0, 0)),
        ],
        out_specs=[
            pl.BlockSpec((_BLK, _H * _D), lambda i: (i, 0)),
            pl.BlockSpec((_BLK, 2 * _H), lambda i: (i, 0)),
        ],
        out_shape=[
            jax.ShapeDtypeStruct((_N, _H * _D), jnp.float32),
            jax.ShapeDtypeStruct((_N, 2 * _H), jnp.float32),
        ],
    )(x, W_gat, M)


def _tail_body(agg_ref, x_ref, bgat_ref, wc_ref, bc_ref, wih_ref, whh_ref,
               bih_ref, bhh_ref, wo_ref, bo_ref, out_ref):
    m = jnp.maximum(agg_ref[...] + bgat_ref[...], 0.0)
    m = lax.dot_general(m, wc_ref[...], (((1,), (1,)), ((), ())),
                        preferred_element_type=jnp.float32)
    m = jnp.maximum(m + bc_ref[...], 0.0)
    x = x_ref[...]
    gi = lax.dot_general(m, wih_ref[...], (((1,), (1,)), ((), ())),
                         preferred_element_type=jnp.float32) + bih_ref[...]
    gh = lax.dot_general(x, whh_ref[...], (((1,), (1,)), ((), ())),
                         preferred_element_type=jnp.float32) + bhh_ref[...]
    i_r = gi[:, :_D]
    i_z = gi[:, _D:2 * _D]
    i_n = gi[:, 2 * _D:]
    h_r = gh[:, :_D]
    h_z = gh[:, _D:2 * _D]
    h_n = gh[:, 2 * _D:]
    r = jax.nn.sigmoid(i_r + h_r)
    z = jax.nn.sigmoid(i_z + h_z)
    n = jnp.tanh(i_n + r * h_n)
    h = jnp.tanh((1.0 - z) * n + z * x)
    out_ref[...] = lax.dot_general(h, wo_ref[...], (((1,), (1,)), ((), ())),
                                   preferred_element_type=jnp.float32) + bo_ref[...]


def _tail(agg, x, b_gat, W_comp, b_comp, W_ih, W_hh, b_ih, b_hh, W_opt, b_opt):
    grid = (_N // _BLK,)
    row = lambda i: (i, 0)
    whole = lambda i: (0, 0)
    return pl.pallas_call(
        _tail_body,
        grid=grid,
        in_specs=[
            pl.BlockSpec((_BLK, _H * _D), row),
            pl.BlockSpec((_BLK, _D), row),
            pl.BlockSpec((1, _H * _D), whole),
            pl.BlockSpec((_D, _H * _D), whole),
            pl.BlockSpec((1, _D), whole),
            pl.BlockSpec((3 * _D, _D), whole),
            pl.BlockSpec((3 * _D, _D), whole),
            pl.BlockSpec((1, 3 * _D), whole),
            pl.BlockSpec((1, 3 * _D), whole),
            pl.BlockSpec((_D, _D), whole),
            pl.BlockSpec((1, _D), whole),
        ],
        out_specs=pl.BlockSpec((_BLK, _D), row),
        out_shape=jax.ShapeDtypeStruct((_N, _D), jnp.float32),
    )(agg, x, b_gat.reshape(1, -1), W_comp, b_comp.reshape(1, -1),
      W_ih, W_hh, b_ih.reshape(1, -1), b_hh.reshape(1, -1),
      W_opt, b_opt.reshape(1, -1))


def _edge_phase(xp, a, src, dst):
    asrc = a[:, :_H]
    adst = a[:, _H:]
    alpha = asrc[src] + adst[dst]
    alpha = jnp.where(alpha >= 0, alpha, 0.2 * alpha)
    w = jnp.exp(alpha)
    denom = jax.ops.segment_sum(w, dst, num_segments=_N)
    wide = jnp.repeat(w, _D, axis=1)
    msg = xp[src] * wide
    out = jax.ops.segment_sum(msg, dst, num_segments=_N)
    return out / jnp.repeat(denom, _D, axis=1)


def kernel(x, edge_index, W_gat, att_src, att_dst, b_gat, W_comp, b_comp,
           W_ih, W_hh, b_ih, b_hh, W_opt, b_opt):
    eye = jnp.eye(_H, dtype=jnp.float32)
    m_src = (eye[:, None, :] * att_src[:, :, None]).reshape(_H * _D, _H)
    m_dst = (eye[:, None, :] * att_dst[:, :, None]).reshape(_H * _D, _H)
    M = jnp.concatenate([m_src, m_dst], axis=1)

    loop = jnp.arange(_N, dtype=edge_index.dtype)
    src = jnp.concatenate([edge_index[0], loop])
    dst = jnp.concatenate([edge_index[1], loop])

    xp, a = _proj(x, W_gat, M)
    agg = _edge_phase(xp, a, src, dst)
    return _tail(agg, x, b_gat, W_comp, b_comp, W_ih, W_hh, b_ih, b_hh,
                 W_opt, b_opt)


# speedup vs baseline: 6.3754x; 1.0647x over previous
import functools

import jax
import jax.numpy as jnp
from jax import lax
from jax.experimental import pallas as pl
from jax.experimental.pallas import tpu as pltpu
from jax.experimental.pallas import tpu_sc as plsc

_N = 10000
_D = 128
_H = 8
_E = 320000
_BLK = 400

_NW = 32
_CT = 80
_NPASS = 4
_SCAN = 3072
_EP = 331776
_NBLK = _EP // _SCAN
_K = 32
_MCAP = _SCAN + _K


def _proj_body(x_ref, wt_ref, m_ref, xp_ref, a_ref):
    x = x_ref[...]
    xp = lax.dot_general(x, wt_ref[...], (((1,), (1,)), ((), ())),
                         preferred_element_type=jnp.float32)
    xp_ref[...] = xp
    a_ref[...] = jnp.dot(xp, m_ref[...], preferred_element_type=jnp.float32)


def _proj(x, W_gat, M):
    return pl.pallas_call(
        _proj_body,
        grid=(_N // _BLK,),
        in_specs=[
            pl.BlockSpec((_BLK, _D), lambda i: (i, 0)),
            pl.BlockSpec((_H * _D, _D), lambda i: (0, 0)),
            pl.BlockSpec((_H * _D, 2 * _H), lambda i: (0, 0)),
        ],
        out_specs=[
            pl.BlockSpec((_BLK, _H * _D), lambda i: (i, 0)),
            pl.BlockSpec((_BLK, 2 * _H), lambda i: (i, 0)),
        ],
        out_shape=[
            jax.ShapeDtypeStruct((_N, _H * _D), jnp.float32),
            jax.ShapeDtypeStruct((_N, 2 * _H), jnp.float32),
        ],
    )(x, W_gat, M)


def _i16(v):
    return jnp.full((16,), v, dtype=jnp.int32)


def _f16(v):
    return jnp.full((16,), v, dtype=jnp.float32)


def _edge_body(src_hbm, dst_hbm, asrc_hbm, adst_hbm, xp_hbm, out_hbm,
               dst_buf, src_buf, msrc, mdst, bsrc, bdst,
               asrc_b, adst_b, w_b, rows_b, acc, den,
               sem0, sem1, sem2):
    iota = lax.iota(jnp.int32, 16)
    wid = lax.axis_index("s") * 2 + lax.axis_index("c")

    def flush(off, nv, gbase):
        for t in range(2):
            lane = iota + (16 * t)
            g = _i16(off) + lane
            sv = plsc.load_gather(msrc, [g])
            dv = plsc.load_gather(mdst, [g])
            valid = lane < _i16(nv)
            sv = jnp.where(valid, sv, 0)
            dv = jnp.where(valid, dv, 0)
            plsc.store_scatter(bsrc, [lane], sv)
            plsc.store_scatter(bdst, [lane], dv)
        c1 = pltpu.async_copy(asrc_hbm.at[bsrc], asrc_b, sem0)
        c2 = pltpu.async_copy(adst_hbm.at[bdst], adst_b, sem1)
        c3 = pltpu.async_copy(xp_hbm.at[bsrc], rows_b, sem2)
        c1.wait()
        c2.wait()
        c3.wait()
        for i in range(16):
            gid = iota + (16 * i)
            ridx = gid >> 3
            cidx = gid & 7
            al = (plsc.load_gather(asrc_b, [ridx, cidx]) +
                  plsc.load_gather(adst_b, [ridx, cidx]))
            al = jnp.where(al >= 0.0, al, al * 0.2)
            w_b[pl.ds(i * 16, 16)] = jnp.exp(al)

        def edge_one(k, carry):
            dsp = plsc.load_gather(bdst, [_i16(k)]) - _i16(gbase)
            for h in range(8):
                ws = plsc.load_gather(w_b, [_i16(k * 8 + h)])
                for j in range(8):
                    cc = iota + (h * 128 + j * 16)
                    rv = plsc.load_gather(rows_b, [_i16(k), cc])
                    plsc.addupdate_scatter(acc, [dsp, cc], rv * ws)
            wrow = plsc.load_gather(w_b, [_i16(k * 8) + iota])
            plsc.addupdate_scatter(den, [dsp, iota], wrow, mask=iota < 8)
            return carry

        lax.fori_loop(0, nv, edge_one, 0)

    def run_pass(p, carry):
        gbase = p * (_NW * _CT) + wid * _CT

        def zero_row(i, c):
            for j in range(64):
                cc = iota + (j * 16)
                plsc.store_scatter(acc, [_i16(i), cc], _f16(0.0))
            plsc.store_scatter(den, [_i16(i), iota], _f16(0.0),
                               mask=iota < 8)
            return c

        lax.fori_loop(0, _CT, zero_row, 0)

        def scan_block(b, cnt):
            cd = pltpu.async_copy(dst_hbm.at[pl.ds(b * _SCAN, _SCAN)],
                                  dst_buf, sem0)
            cs = pltpu.async_copy(src_hbm.at[pl.ds(b * _SCAN, _SCAN)],
                                  src_buf, sem1)
            cd.wait()
            cs.wait()

            def scan_vec(v, cnt):
                d16 = plsc.load_gather(dst_buf, [_i16(v * 16) + iota])
                m = (d16 >= _i16(gbase)) & (d16 < _i16(gbase + _CT))
                mi = m.astype(jnp.int32)
                nm = jnp.sum(mi)

                @pl.when(nm > 0)
                def _():
                    s16 = plsc.load_gather(src_buf, [_i16(v * 16) + iota])
                    pos = _i16(cnt) + plsc.cumsum(mi) - 1
                    plsc.store_scatter(mdst, [pos], d16, mask=m)
                    plsc.store_scatter(msrc, [pos], s16, mask=m)

                return cnt + nm

            cnt = lax.fori_loop(0, _SCAN // 16, scan_vec, cnt)

            def fcond(st):
                off, c = st
                return c - off >= _K

            def fbody(st):
                off, c = st
                flush(off, _K, gbase)
                return (off + _K, c)

            off, cnt = lax.while_loop(fcond, fbody, (0, cnt))

            for t in range(2):
                lane = iota + (16 * t)
                g = _i16(off) + lane
                sv = plsc.load_gather(msrc, [g])
                dv = plsc.load_gather(mdst, [g])
                plsc.store_scatter(msrc, [lane], sv)
                plsc.store_scatter(mdst, [lane], dv)
            return cnt - off

        rem = lax.fori_loop(0, _NBLK, scan_block, 0)
        flush(0, rem, gbase)

        @pl.when(gbase + _CT <= _N)
        def _():
            def nrm_row(i, c):
                si = _i16(i)
                for h in range(8):
                    dv = plsc.load_gather(den, [si, _i16(h)])
                    rs = 1.0 / dv
                    for j in range(8):
                        cc = iota + (h * 128 + j * 16)
                        v = plsc.load_gather(acc, [si, cc])
                        plsc.store_scatter(acc, [si, cc], v * rs)
                return c

            lax.fori_loop(0, _CT, nrm_row, 0)
            pltpu.sync_copy(acc, out_hbm.at[pl.ds(gbase, _CT)])

        return carry

    lax.fori_loop(0, _NPASS, run_pass, 0)


_edge_call = functools.partial(
    pl.kernel,
    _edge_body,
    out_type=jax.ShapeDtypeStruct((_N, _H * _D), jnp.float32),
    mesh=plsc.VectorSubcoreMesh(core_axis_name="c", subcore_axis_name="s"),
    compiler_params=pltpu.CompilerParams(use_tc_tiling_on_sc=False,
                                         needs_layout_passes=False),
    scratch_types=[
        pltpu.VMEM((_SCAN,), jnp.int32),
        pltpu.VMEM((_SCAN,), jnp.int32),
        pltpu.VMEM((_MCAP,), jnp.int32),
        pltpu.VMEM((_MCAP,), jnp.int32),
        pltpu.VMEM((_K,), jnp.int32),
        pltpu.VMEM((_K,), jnp.int32),
        pltpu.VMEM((_K, _H), jnp.float32),
        pltpu.VMEM((_K, _H), jnp.float32),
        pltpu.VMEM((_K * _H,), jnp.float32),
        pltpu.VMEM((_K, _H * _D), jnp.float32),
        pltpu.VMEM((_CT, _H * _D), jnp.float32),
        pltpu.VMEM((_CT, _H), jnp.float32),
        pltpu.SemaphoreType.DMA,
        pltpu.SemaphoreType.DMA,
        pltpu.SemaphoreType.DMA,
    ],
)()


def _tail_body(agg_ref, x_ref, bgat_ref, wc_ref, bc_ref, wih_ref, whh_ref,
               bih_ref, bhh_ref, wo_ref, bo_ref, out_ref):
    m = jnp.maximum(agg_ref[...] + bgat_ref[...], 0.0)
    m = lax.dot_general(m, wc_ref[...], (((1,), (1,)), ((), ())),
                        preferred_element_type=jnp.float32)
    m = jnp.maximum(m + bc_ref[...], 0.0)
    x = x_ref[...]
    gi = lax.dot_general(m, wih_ref[...], (((1,), (1,)), ((), ())),
                         preferred_element_type=jnp.float32) + bih_ref[...]
    gh = lax.dot_general(x, whh_ref[...], (((1,), (1,)), ((), ())),
                         preferred_element_type=jnp.float32) + bhh_ref[...]
    i_r = gi[:, :_D]
    i_z = gi[:, _D:2 * _D]
    i_n = gi[:, 2 * _D:]
    h_r = gh[:, :_D]
    h_z = gh[:, _D:2 * _D]
    h_n = gh[:, 2 * _D:]
    r = jax.nn.sigmoid(i_r + h_r)
    z = jax.nn.sigmoid(i_z + h_z)
    n = jnp.tanh(i_n + r * h_n)
    h = jnp.tanh((1.0 - z) * n + z * x)
    out_ref[...] = lax.dot_general(h, wo_ref[...], (((1,), (1,)), ((), ())),
                                   preferred_element_type=jnp.float32) + bo_ref[...]


def _tail(agg, x, b_gat, W_comp, b_comp, W_ih, W_hh, b_ih, b_hh, W_opt, b_opt):
    row = lambda i: (i, 0)
    whole = lambda i: (0, 0)
    return pl.pallas_call(
        _tail_body,
        grid=(_N // _BLK,),
        in_specs=[
            pl.BlockSpec((_BLK, _H * _D), row),
            pl.BlockSpec((_BLK, _D), row),
            pl.BlockSpec((1, _H * _D), whole),
            pl.BlockSpec((_D, _H * _D), whole),
            pl.BlockSpec((1, _D), whole),
            pl.BlockSpec((3 * _D, _D), whole),
            pl.BlockSpec((3 * _D, _D), whole),
            pl.BlockSpec((1, 3 * _D), whole),
            pl.BlockSpec((1, 3 * _D), whole),
            pl.BlockSpec((_D, _D), whole),
            pl.BlockSpec((1, _D), whole),
        ],
        out_specs=pl.BlockSpec((_BLK, _D), row),
        out_shape=jax.ShapeDtypeStruct((_N, _D), jnp.float32),
    )(agg, x, b_gat.reshape(1, -1), W_comp, b_comp.reshape(1, -1),
      W_ih, W_hh, b_ih.reshape(1, -1), b_hh.reshape(1, -1),
      W_opt, b_opt.reshape(1, -1))


def kernel(x, edge_index, W_gat, att_src, att_dst, b_gat, W_comp, b_comp,
           W_ih, W_hh, b_ih, b_hh, W_opt, b_opt):
    eye = jnp.eye(_H, dtype=jnp.float32)
    m_src = (eye[:, None, :] * att_src[:, :, None]).reshape(_H * _D, _H)
    m_dst = (eye[:, None, :] * att_dst[:, :, None]).reshape(_H * _D, _H)
    M = jnp.concatenate([m_src, m_dst], axis=1)

    loop = jnp.arange(_N, dtype=edge_index.dtype)
    pad = _EP - (_E + _N)
    src = jnp.concatenate([edge_index[0], loop,
                           jnp.zeros((pad,), edge_index.dtype)])
    dst = jnp.concatenate([edge_index[1], loop,
                           jnp.full((pad,), 1 << 30, edge_index.dtype)])

    xp, a = _proj(x, W_gat, M)
    agg = _edge_call(src, dst, a[:, :_H], a[:, _H:], xp)
    return _tail(agg, x, b_gat, W_comp, b_comp, W_ih, W_hh, b_ih, b_hh,
                 W_opt, b_opt)


# speedup vs baseline: 7.7501x; 1.2156x over previous
import functools

import jax
import jax.numpy as jnp
from jax import lax
from jax.experimental import pallas as pl
from jax.experimental.pallas import tpu as pltpu
from jax.experimental.pallas import tpu_sc as plsc

_N = 10000
_D = 128
_H = 8
_E = 320000
_BLK = 400

_NW = 32
_CT = 80
_NPASS = 4
_SCAN = 2048
_EP = 331776
_NBLK = _EP // _SCAN
_K = 32
_MCAP = _SCAN + _K


def _proj_body(x_ref, wt_ref, m_ref, xp_ref, a_ref):
    x = x_ref[...]
    xp = lax.dot_general(x, wt_ref[...], (((1,), (1,)), ((), ())),
                         preferred_element_type=jnp.float32)
    xp_ref[...] = xp
    a_ref[...] = jnp.dot(xp, m_ref[...], preferred_element_type=jnp.float32)


def _proj(x, W_gat, M):
    return pl.pallas_call(
        _proj_body,
        grid=(_N // _BLK,),
        in_specs=[
            pl.BlockSpec((_BLK, _D), lambda i: (i, 0)),
            pl.BlockSpec((_H * _D, _D), lambda i: (0, 0)),
            pl.BlockSpec((_H * _D, 2 * _H), lambda i: (0, 0)),
        ],
        out_specs=[
            pl.BlockSpec((_BLK, _H * _D), lambda i: (i, 0)),
            pl.BlockSpec((_BLK, 2 * _H), lambda i: (i, 0)),
        ],
        out_shape=[
            jax.ShapeDtypeStruct((_N, _H * _D), jnp.float32),
            jax.ShapeDtypeStruct((_N, 2 * _H), jnp.float32),
        ],
    )(x, W_gat, M)


def _i16(v):
    return jnp.full((16,), v, dtype=jnp.int32)


def _f16(v):
    return jnp.full((16,), v, dtype=jnp.float32)


def _edge_body(src_hbm, dst_hbm, asrc_hbm, adst_hbm, xp_hbm, out_hbm,
               dbuf0, sbuf0, dbuf1, sbuf1, msrc, mdst, bsrc, bdst,
               asrc_b, adst_local, w_b, rows_b, acc, den,
               semd0, sems0, semd1, sems1, semf0, semf1):
    iota = lax.iota(jnp.int32, 16)
    wid = lax.axis_index("s") * 2 + lax.axis_index("c")

    def flush(off, nv, gbase):
        for t in range(2):
            lane = iota + (16 * t)
            g = _i16(off) + lane
            sv = plsc.load_gather(msrc, [g])
            dv = plsc.load_gather(mdst, [g])
            valid = lane < _i16(nv)
            sv = jnp.where(valid, sv, 0)
            dv = jnp.where(valid, dv, _i16(gbase))
            plsc.store_scatter(bsrc, [lane], sv)
            plsc.store_scatter(bdst, [lane], dv)
        c1 = pltpu.async_copy(asrc_hbm.at[bsrc], asrc_b, semf0)
        c3 = pltpu.async_copy(xp_hbm.at[bsrc], rows_b, semf1)
        c1.wait()
        c3.wait()
        for i in range(16):
            gid = iota + (16 * i)
            ridx = gid >> 3
            cidx = gid & 7
            dl = plsc.load_gather(bdst, [ridx]) - _i16(gbase)
            al = (plsc.load_gather(asrc_b, [ridx, cidx]) +
                  plsc.load_gather(adst_local, [dl, cidx]))
            al = jnp.where(al >= 0.0, al, al * 0.2)
            w_b[pl.ds(i * 16, 16)] = jnp.exp(al)

        def edge_one(k, carry):
            dsp = plsc.load_gather(bdst, [_i16(k)]) - _i16(gbase)
            for h in range(8):
                ws = plsc.load_gather(w_b, [_i16(k * 8 + h)])
                for j in range(8):
                    cc = iota + (h * 128 + j * 16)
                    rv = plsc.load_gather(rows_b, [_i16(k), cc])
                    plsc.addupdate_scatter(acc, [dsp, cc], rv * ws)
            wrow = plsc.load_gather(w_b, [_i16(k * 8) + iota])
            plsc.addupdate_scatter(den, [dsp, iota], wrow, mask=iota < 8)
            return carry

        lax.fori_loop(0, nv, edge_one, 0)

    def run_pass(p, carry):
        gbase = p * (_NW * _CT) + wid * _CT

        def zero_row(i, c):
            for j in range(64):
                cc = iota + (j * 16)
                plsc.store_scatter(acc, [_i16(i), cc], _f16(0.0))
            plsc.store_scatter(den, [_i16(i), iota], _f16(0.0),
                               mask=iota < 8)
            return c

        lax.fori_loop(0, _CT, zero_row, 0)

        cb = jnp.minimum(gbase, _N - _CT)
        pltpu.sync_copy(adst_hbm.at[pl.ds(cb, _CT)], adst_local)

        def scan_chunk(dbuf, sbuf, b, cnt_v):
            lo = _i16(gbase)
            hi = _i16(gbase + _CT)

            def scan_vec(v, cnt_v):
                base = _i16(v * 16) + iota
                d16 = plsc.load_gather(dbuf, [base])
                m = (d16 >= lo) & (d16 < hi)
                mi = m.astype(jnp.int32)
                pc = plsc.all_reduce_population_count(m)
                pos = cnt_v + plsc.cumsum(mi) - 1
                s16 = plsc.load_gather(sbuf, [base])
                plsc.store_scatter(mdst, [pos], d16, mask=m)
                plsc.store_scatter(msrc, [pos], s16, mask=m)
                return cnt_v + pc

            cnt_v = lax.fori_loop(0, _SCAN // 16, scan_vec, cnt_v)
            cnt = jnp.max(cnt_v)

            def fcond(st):
                off, c = st
                return c - off >= _K

            def fbody(st):
                off, c = st
                flush(off, _K, gbase)
                return (off + _K, c)

            off, cnt = lax.while_loop(fcond, fbody, (0, cnt))

            for t in range(2):
                lane = iota + (16 * t)
                g = _i16(off) + lane
                sv = plsc.load_gather(msrc, [g])
                dv = plsc.load_gather(mdst, [g])
                plsc.store_scatter(msrc, [lane], sv)
                plsc.store_scatter(mdst, [lane], dv)
            return _i16(cnt - off)

        def issue(b, dbuf, sbuf, semd, sems):
            cd = pltpu.async_copy(dst_hbm.at[pl.ds(b * _SCAN, _SCAN)],
                                  dbuf, semd)
            cs = pltpu.async_copy(src_hbm.at[pl.ds(b * _SCAN, _SCAN)],
                                  sbuf, sems)
            return cd, cs

        issue(0, dbuf0, sbuf0, semd0, sems0)

        def block2(i, cnt_v):
            b0 = 2 * i
            issue(b0 + 1, dbuf1, sbuf1, semd1, sems1)
            pltpu.make_async_copy(dst_hbm.at[pl.ds(b0 * _SCAN, _SCAN)],
                                  dbuf0, semd0).wait()
            pltpu.make_async_copy(src_hbm.at[pl.ds(b0 * _SCAN, _SCAN)],
                                  sbuf0, sems0).wait()
            cnt_v = scan_chunk(dbuf0, sbuf0, b0, cnt_v)
            nxt = jnp.minimum(b0 + 2, _NBLK - 1)
            issue(nxt, dbuf0, sbuf0, semd0, sems0)
            pltpu.make_async_copy(dst_hbm.at[pl.ds(b0 * _SCAN, _SCAN)],
                                  dbuf1, semd1).wait()
            pltpu.make_async_copy(src_hbm.at[pl.ds(b0 * _SCAN, _SCAN)],
                                  sbuf1, sems1).wait()
            cnt_v = scan_chunk(dbuf1, sbuf1, b0 + 1, cnt_v)
            return cnt_v

        cnt_v = lax.fori_loop(0, _NBLK // 2, block2, _i16(0))
        pltpu.make_async_copy(dst_hbm.at[pl.ds(0, _SCAN)], dbuf0, semd0).wait()
        pltpu.make_async_copy(src_hbm.at[pl.ds(0, _SCAN)], sbuf0, sems0).wait()

        rem = jnp.max(cnt_v)
        flush(0, rem, gbase)

        @pl.when(gbase + _CT <= _N)
        def _():
            def nrm_row(i, c):
                si = _i16(i)
                for h in range(8):
                    dv = plsc.load_gather(den, [si, _i16(h)])
                    rs = 1.0 / dv
                    for j in range(8):
                        cc = iota + (h * 128 + j * 16)
                        v = plsc.load_gather(acc, [si, cc])
                        plsc.store_scatter(acc, [si, cc], v * rs)
                return c

            lax.fori_loop(0, _CT, nrm_row, 0)
            pltpu.sync_copy(acc, out_hbm.at[pl.ds(gbase, _CT)])

        return carry

    lax.fori_loop(0, _NPASS, run_pass, 0)


_edge_call = functools.partial(
    pl.kernel,
    _edge_body,
    out_type=jax.ShapeDtypeStruct((_N, _H * _D), jnp.float32),
    mesh=plsc.VectorSubcoreMesh(core_axis_name="c", subcore_axis_name="s"),
    compiler_params=pltpu.CompilerParams(use_tc_tiling_on_sc=False,
                                         needs_layout_passes=False),
    scratch_types=[
        pltpu.VMEM((_SCAN,), jnp.int32),
        pltpu.VMEM((_SCAN,), jnp.int32),
        pltpu.VMEM((_SCAN,), jnp.int32),
        pltpu.VMEM((_SCAN,), jnp.int32),
        pltpu.VMEM((_MCAP,), jnp.int32),
        pltpu.VMEM((_MCAP,), jnp.int32),
        pltpu.VMEM((_K,), jnp.int32),
        pltpu.VMEM((_K,), jnp.int32),
        pltpu.VMEM((_K, _H), jnp.float32),
        pltpu.VMEM((_CT, _H), jnp.float32),
        pltpu.VMEM((_K * _H,), jnp.float32),
        pltpu.VMEM((_K, _H * _D), jnp.float32),
        pltpu.VMEM((_CT, _H * _D), jnp.float32),
        pltpu.VMEM((_CT, _H), jnp.float32),
        pltpu.SemaphoreType.DMA,
        pltpu.SemaphoreType.DMA,
        pltpu.SemaphoreType.DMA,
        pltpu.SemaphoreType.DMA,
        pltpu.SemaphoreType.DMA,
        pltpu.SemaphoreType.DMA,
    ],
)()


def _tail_body(agg_ref, x_ref, bgat_ref, wc_ref, bc_ref, wih_ref, whh_ref,
               bih_ref, bhh_ref, wo_ref, bo_ref, out_ref):
    m = jnp.maximum(agg_ref[...] + bgat_ref[...], 0.0)
    m = lax.dot_general(m, wc_ref[...], (((1,), (1,)), ((), ())),
                        preferred_element_type=jnp.float32)
    m = jnp.maximum(m + bc_ref[...], 0.0)
    x = x_ref[...]
    gi = lax.dot_general(m, wih_ref[...], (((1,), (1,)), ((), ())),
                         preferred_element_type=jnp.float32) + bih_ref[...]
    gh = lax.dot_general(x, whh_ref[...], (((1,), (1,)), ((), ())),
                         preferred_element_type=jnp.float32) + bhh_ref[...]
    i_r = gi[:, :_D]
    i_z = gi[:, _D:2 * _D]
    i_n = gi[:, 2 * _D:]
    h_r = gh[:, :_D]
    h_z = gh[:, _D:2 * _D]
    h_n = gh[:, 2 * _D:]
    r = jax.nn.sigmoid(i_r + h_r)
    z = jax.nn.sigmoid(i_z + h_z)
    n = jnp.tanh(i_n + r * h_n)
    h = jnp.tanh((1.0 - z) * n + z * x)
    out_ref[...] = lax.dot_general(h, wo_ref[...], (((1,), (1,)), ((), ())),
                                   preferred_element_type=jnp.float32) + bo_ref[...]


def _tail(agg, x, b_gat, W_comp, b_comp, W_ih, W_hh, b_ih, b_hh, W_opt, b_opt):
    row = lambda i: (i, 0)
    whole = lambda i: (0, 0)
    return pl.pallas_call(
        _tail_body,
        grid=(_N // _BLK,),
        in_specs=[
            pl.BlockSpec((_BLK, _H * _D), row),
            pl.BlockSpec((_BLK, _D), row),
            pl.BlockSpec((1, _H * _D), whole),
            pl.BlockSpec((_D, _H * _D), whole),
            pl.BlockSpec((1, _D), whole),
            pl.BlockSpec((3 * _D, _D), whole),
            pl.BlockSpec((3 * _D, _D), whole),
            pl.BlockSpec((1, 3 * _D), whole),
            pl.BlockSpec((1, 3 * _D), whole),
            pl.BlockSpec((_D, _D), whole),
            pl.BlockSpec((1, _D), whole),
        ],
        out_specs=pl.BlockSpec((_BLK, _D), row),
        out_shape=jax.ShapeDtypeStruct((_N, _D), jnp.float32),
    )(agg, x, b_gat.reshape(1, -1), W_comp, b_comp.reshape(1, -1),
      W_ih, W_hh, b_ih.reshape(1, -1), b_hh.reshape(1, -1),
      W_opt, b_opt.reshape(1, -1))


def kernel(x, edge_index, W_gat, att_src, att_dst, b_gat, W_comp, b_comp,
           W_ih, W_hh, b_ih, b_hh, W_opt, b_opt):
    eye = jnp.eye(_H, dtype=jnp.float32)
    m_src = (eye[:, None, :] * att_src[:, :, None]).reshape(_H * _D, _H)
    m_dst = (eye[:, None, :] * att_dst[:, :, None]).reshape(_H * _D, _H)
    M = jnp.concatenate([m_src, m_dst], axis=1)

    loop = jnp.arange(_N, dtype=edge_index.dtype)
    pad = _EP - (_E + _N)
    src = jnp.concatenate([edge_index[0], loop,
                           jnp.zeros((pad,), edge_index.dtype)])
    dst = jnp.concatenate([edge_index[1], loop,
                           jnp.full((pad,), 1 << 30, edge_index.dtype)])

    xp, a = _proj(x, W_gat, M)
    agg = _edge_call(src, dst, a[:, :_H], a[:, _H:], xp)
    return _tail(agg, x, b_gat, W_comp, b_comp, W_ih, W_hh, b_ih, b_hh,
                 W_opt, b_opt)


# speedup vs baseline: 8.0313x; 1.0363x over previous
import functools

import jax
import jax.numpy as jnp
from jax import lax
from jax.experimental import pallas as pl
from jax.experimental.pallas import tpu as pltpu
from jax.experimental.pallas import tpu_sc as plsc

_N = 10000
_D = 128
_H = 8
_E = 320000
_BLK = 400

_NW = 32
_CT = 80
_NPASS = 4
_SCAN = 2048
_EP = 331776
_NBLK = _EP // _SCAN
_K = 32
_MCAP = _SCAN + _K


def _proj_body(x_ref, wt_ref, m_ref, xp_ref, a_ref):
    x = x_ref[...]
    xp = lax.dot_general(x, wt_ref[...], (((1,), (1,)), ((), ())),
                         preferred_element_type=jnp.float32)
    xp_ref[...] = xp
    a_ref[...] = jnp.dot(xp, m_ref[...], preferred_element_type=jnp.float32)


def _proj(x, W_gat, M):
    return pl.pallas_call(
        _proj_body,
        grid=(_N // _BLK,),
        in_specs=[
            pl.BlockSpec((_BLK, _D), lambda i: (i, 0)),
            pl.BlockSpec((_H * _D, _D), lambda i: (0, 0)),
            pl.BlockSpec((_H * _D, 2 * _H), lambda i: (0, 0)),
        ],
        out_specs=[
            pl.BlockSpec((_BLK, _H * _D), lambda i: (i, 0)),
            pl.BlockSpec((_BLK, 2 * _H), lambda i: (i, 0)),
        ],
        out_shape=[
            jax.ShapeDtypeStruct((_N, _H * _D), jnp.float32),
            jax.ShapeDtypeStruct((_N, 2 * _H), jnp.float32),
        ],
    )(x, W_gat, M)


def _i16(v):
    return jnp.full((16,), v, dtype=jnp.int32)


def _f16(v):
    return jnp.full((16,), v, dtype=jnp.float32)


def _edge_body(src_hbm, dst_hbm, asrc_hbm, adst_hbm, xp_hbm, out_hbm,
               dbuf0, sbuf0, dbuf1, sbuf1, msrc, mdst, bsrc, bdst,
               asrc_b, adst_local, w_b, rows_b, acc, den,
               semd0, sems0, semd1, sems1, semf0, semf1, semf2):
    iota = lax.iota(jnp.int32, 16)
    wid = lax.axis_index("s") * 2 + lax.axis_index("c")

    def flush(off, nv, gbase):
        for t in range(2):
            lane = iota + (16 * t)
            g = _i16(off) + lane
            sv = plsc.load_gather(msrc, [g])
            dv = plsc.load_gather(mdst, [g])
            valid = lane < _i16(nv)
            sv = jnp.where(valid, sv, 0)
            dv = jnp.where(valid, dv, _i16(gbase))
            plsc.store_scatter(bsrc, [lane], sv)
            plsc.store_scatter(bdst, [lane], dv)
        ca = pltpu.async_copy(asrc_hbm.at[bsrc], asrc_b, semf0)
        c0 = pltpu.async_copy(xp_hbm.at[bsrc.at[pl.ds(0, 8)]],
                              rows_b.at[pl.ds(0, 8)], semf1)
        c1 = pltpu.async_copy(xp_hbm.at[bsrc.at[pl.ds(8, 8)]],
                              rows_b.at[pl.ds(8, 8)], semf2)
        ca.wait()
        for i in range(16):
            gid = iota + (16 * i)
            ridx = gid >> 3
            cidx = gid & 7
            dl = plsc.load_gather(bdst, [ridx]) - _i16(gbase)
            al = (plsc.load_gather(asrc_b, [ridx, cidx]) +
                  plsc.load_gather(adst_local, [dl, cidx]))
            al = jnp.where(al >= 0.0, al, al * 0.2)
            w_b[pl.ds(i * 16, 16)] = jnp.exp(al)

        def edge_one(k, carry):
            dsp = plsc.load_gather(bdst, [_i16(k)]) - _i16(gbase)
            for h in range(8):
                ws = plsc.load_gather(w_b, [_i16(k * 8 + h)])
                for j in range(8):
                    cc = iota + (h * 128 + j * 16)
                    rv = plsc.load_gather(rows_b, [_i16(k), cc])
                    plsc.addupdate_scatter(acc, [dsp, cc], rv * ws)
            wrow = plsc.load_gather(w_b, [_i16(k * 8) + iota])
            plsc.addupdate_scatter(den, [dsp, iota], wrow, mask=iota < 8)
            return carry

        c0.wait()
        c2 = pltpu.async_copy(xp_hbm.at[bsrc.at[pl.ds(16, 8)]],
                              rows_b.at[pl.ds(16, 8)], semf1)
        lax.fori_loop(0, jnp.minimum(nv, 8), edge_one, 0)
        c1.wait()
        c3 = pltpu.async_copy(xp_hbm.at[bsrc.at[pl.ds(24, 8)]],
                              rows_b.at[pl.ds(24, 8)], semf2)
        lax.fori_loop(8, jnp.clip(nv, 8, 16), edge_one, 0)
        c2.wait()
        lax.fori_loop(16, jnp.clip(nv, 16, 24), edge_one, 0)
        c3.wait()
        lax.fori_loop(24, jnp.clip(nv, 24, 32), edge_one, 0)

    def run_pass(p, carry):
        gbase = p * (_NW * _CT) + wid * _CT

        def zero_row(i, c):
            for j in range(64):
                cc = iota + (j * 16)
                plsc.store_scatter(acc, [_i16(i), cc], _f16(0.0))
            plsc.store_scatter(den, [_i16(i), iota], _f16(0.0),
                               mask=iota < 8)
            return c

        lax.fori_loop(0, _CT, zero_row, 0)

        cb = jnp.minimum(gbase, _N - _CT)
        pltpu.sync_copy(adst_hbm.at[pl.ds(cb, _CT)], adst_local)

        def scan_chunk(dbuf, sbuf, b, cnt_v):
            lo = _i16(gbase)
            hi = _i16(gbase + _CT)

            def scan_vec(v, cnt_v):
                base = _i16(v * 16) + iota
                d16 = plsc.load_gather(dbuf, [base])
                m = (d16 >= lo) & (d16 < hi)
                mi = m.astype(jnp.int32)
                pc = plsc.all_reduce_population_count(m)
                pos = cnt_v + plsc.cumsum(mi) - 1
                s16 = plsc.load_gather(sbuf, [base])
                plsc.store_scatter(mdst, [pos], d16, mask=m)
                plsc.store_scatter(msrc, [pos], s16, mask=m)
                return cnt_v + pc

            cnt_v = lax.fori_loop(0, _SCAN // 16, scan_vec, cnt_v)
            cnt = jnp.max(cnt_v)

            def fcond(st):
                off, c = st
                return c - off >= _K

            def fbody(st):
                off, c = st
                flush(off, _K, gbase)
                return (off + _K, c)

            off, cnt = lax.while_loop(fcond, fbody, (0, cnt))

            for t in range(2):
                lane = iota + (16 * t)
                g = _i16(off) + lane
                sv = plsc.load_gather(msrc, [g])
                dv = plsc.load_gather(mdst, [g])
                plsc.store_scatter(msrc, [lane], sv)
                plsc.store_scatter(mdst, [lane], dv)
            return _i16(cnt - off)

        def issue(b, dbuf, sbuf, semd, sems):
            cd = pltpu.async_copy(dst_hbm.at[pl.ds(b * _SCAN, _SCAN)],
                                  dbuf, semd)
            cs = pltpu.async_copy(src_hbm.at[pl.ds(b * _SCAN, _SCAN)],
                                  sbuf, sems)
            return cd, cs

        issue(0, dbuf0, sbuf0, semd0, sems0)

        def block2(i, cnt_v):
            b0 = 2 * i
            issue(b0 + 1, dbuf1, sbuf1, semd1, sems1)
            pltpu.make_async_copy(dst_hbm.at[pl.ds(b0 * _SCAN, _SCAN)],
                                  dbuf0, semd0).wait()
            pltpu.make_async_copy(src_hbm.at[pl.ds(b0 * _SCAN, _SCAN)],
                                  sbuf0, sems0).wait()
            cnt_v = scan_chunk(dbuf0, sbuf0, b0, cnt_v)
            nxt = jnp.minimum(b0 + 2, _NBLK - 1)
            issue(nxt, dbuf0, sbuf0, semd0, sems0)
            pltpu.make_async_copy(dst_hbm.at[pl.ds(b0 * _SCAN, _SCAN)],
                                  dbuf1, semd1).wait()
            pltpu.make_async_copy(src_hbm.at[pl.ds(b0 * _SCAN, _SCAN)],
                                  sbuf1, sems1).wait()
            cnt_v = scan_chunk(dbuf1, sbuf1, b0 + 1, cnt_v)
            return cnt_v

        cnt_v = lax.fori_loop(0, _NBLK // 2, block2, _i16(0))
        pltpu.make_async_copy(dst_hbm.at[pl.ds(0, _SCAN)], dbuf0, semd0).wait()
        pltpu.make_async_copy(src_hbm.at[pl.ds(0, _SCAN)], sbuf0, sems0).wait()

        rem = jnp.max(cnt_v)
        flush(0, rem, gbase)

        @pl.when(gbase + _CT <= _N)
        def _():
            def nrm_row(i, c):
                si = _i16(i)
                for h in range(8):
                    dv = plsc.load_gather(den, [si, _i16(h)])
                    rs = 1.0 / dv
                    for j in range(8):
                        cc = iota + (h * 128 + j * 16)
                        v = plsc.load_gather(acc, [si, cc])
                        plsc.store_scatter(acc, [si, cc], v * rs)
                return c

            lax.fori_loop(0, _CT, nrm_row, 0)
            pltpu.sync_copy(acc, out_hbm.at[pl.ds(gbase, _CT)])

        return carry

    lax.fori_loop(0, _NPASS, run_pass, 0)


_edge_call = functools.partial(
    pl.kernel,
    _edge_body,
    out_type=jax.ShapeDtypeStruct((_N, _H * _D), jnp.float32),
    mesh=plsc.VectorSubcoreMesh(core_axis_name="c", subcore_axis_name="s"),
    compiler_params=pltpu.CompilerParams(use_tc_tiling_on_sc=False,
                                         needs_layout_passes=False),
    scratch_types=[
        pltpu.VMEM((_SCAN,), jnp.int32),
        pltpu.VMEM((_SCAN,), jnp.int32),
        pltpu.VMEM((_SCAN,), jnp.int32),
        pltpu.VMEM((_SCAN,), jnp.int32),
        pltpu.VMEM((_MCAP,), jnp.int32),
        pltpu.VMEM((_MCAP,), jnp.int32),
        pltpu.VMEM((_K,), jnp.int32),
        pltpu.VMEM((_K,), jnp.int32),
        pltpu.VMEM((_K, _H), jnp.float32),
        pltpu.VMEM((_CT, _H), jnp.float32),
        pltpu.VMEM((_K * _H,), jnp.float32),
        pltpu.VMEM((_K, _H * _D), jnp.float32),
        pltpu.VMEM((_CT, _H * _D), jnp.float32),
        pltpu.VMEM((_CT, _H), jnp.float32),
        pltpu.SemaphoreType.DMA,
        pltpu.SemaphoreType.DMA,
        pltpu.SemaphoreType.DMA,
        pltpu.SemaphoreType.DMA,
        pltpu.SemaphoreType.DMA,
        pltpu.SemaphoreType.DMA,
        pltpu.SemaphoreType.DMA,
    ],
)()


def _tail_body(agg_ref, x_ref, bgat_ref, wc_ref, bc_ref, wih_ref, whh_ref,
               bih_ref, bhh_ref, wo_ref, bo_ref, out_ref):
    m = jnp.maximum(agg_ref[...] + bgat_ref[...], 0.0)
    m = lax.dot_general(m, wc_ref[...], (((1,), (1,)), ((), ())),
                        preferred_element_type=jnp.float32)
    m = jnp.maximum(m + bc_ref[...], 0.0)
    x = x_ref[...]
    gi = lax.dot_general(m, wih_ref[...], (((1,), (1,)), ((), ())),
                         preferred_element_type=jnp.float32) + bih_ref[...]
    gh = lax.dot_general(x, whh_ref[...], (((1,), (1,)), ((), ())),
                         preferred_element_type=jnp.float32) + bhh_ref[...]
    i_r = gi[:, :_D]
    i_z = gi[:, _D:2 * _D]
    i_n = gi[:, 2 * _D:]
    h_r = gh[:, :_D]
    h_z = gh[:, _D:2 * _D]
    h_n = gh[:, 2 * _D:]
    r = jax.nn.sigmoid(i_r + h_r)
    z = jax.nn.sigmoid(i_z + h_z)
    n = jnp.tanh(i_n + r * h_n)
    h = jnp.tanh((1.0 - z) * n + z * x)
    out_ref[...] = lax.dot_general(h, wo_ref[...], (((1,), (1,)), ((), ())),
                                   preferred_element_type=jnp.float32) + bo_ref[...]


def _tail(agg, x, b_gat, W_comp, b_comp, W_ih, W_hh, b_ih, b_hh, W_opt, b_opt):
    row = lambda i: (i, 0)
    whole = lambda i: (0, 0)
    return pl.pallas_call(
        _tail_body,
        grid=(_N // _BLK,),
        in_specs=[
            pl.BlockSpec((_BLK, _H * _D), row),
            pl.BlockSpec((_BLK, _D), row),
            pl.BlockSpec((1, _H * _D), whole),
            pl.BlockSpec((_D, _H * _D), whole),
            pl.BlockSpec((1, _D), whole),
            pl.BlockSpec((3 * _D, _D), whole),
            pl.BlockSpec((3 * _D, _D), whole),
            pl.BlockSpec((1, 3 * _D), whole),
            pl.BlockSpec((1, 3 * _D), whole),
            pl.BlockSpec((_D, _D), whole),
            pl.BlockSpec((1, _D), whole),
        ],
        out_specs=pl.BlockSpec((_BLK, _D), row),
        out_shape=jax.ShapeDtypeStruct((_N, _D), jnp.float32),
    )(agg, x, b_gat.reshape(1, -1), W_comp, b_comp.reshape(1, -1),
      W_ih, W_hh, b_ih.reshape(1, -1), b_hh.reshape(1, -1),
      W_opt, b_opt.reshape(1, -1))


def kernel(x, edge_index, W_gat, att_src, att_dst, b_gat, W_comp, b_comp,
           W_ih, W_hh, b_ih, b_hh, W_opt, b_opt):
    eye = jnp.eye(_H, dtype=jnp.float32)
    m_src = (eye[:, None, :] * att_src[:, :, None]).reshape(_H * _D, _H)
    m_dst = (eye[:, None, :] * att_dst[:, :, None]).reshape(_H * _D, _H)
    M = jnp.concatenate([m_src, m_dst], axis=1)

    loop = jnp.arange(_N, dtype=edge_index.dtype)
    pad = _EP - (_E + _N)
    src = jnp.concatenate([edge_index[0], loop,
                           jnp.zeros((pad,), edge_index.dtype)])
    dst = jnp.concatenate([edge_index[1], loop,
                           jnp.full((pad,), 1 << 30, edge_index.dtype)])

    xp, a = _proj(x, W_gat, M)
    agg = _edge_call(src, dst, a[:, :_H], a[:, _H:], xp)
    return _tail(agg, x, b_gat, W_comp, b_comp, W_ih, W_hh, b_ih, b_hh,
                 W_opt, b_opt)


# speedup vs baseline: 8.0626x; 1.0039x over previous
import functools

import jax
import jax.numpy as jnp
from jax import lax
from jax.experimental import pallas as pl
from jax.experimental.pallas import tpu as pltpu
from jax.experimental.pallas import tpu_sc as plsc

_N = 10000
_D = 128
_H = 8
_E = 320000
_BLK = 400

_NW = 32
_CT = 80
_NPASS = 4
_SCAN = 2048
_EP = 331776
_NBLK = _EP // _SCAN
_K = 32
_MCAP = _SCAN + _K


def _proj_body(x_ref, wt_ref, m_ref, xp_ref, a_ref):
    x = x_ref[...]
    xp = lax.dot_general(x, wt_ref[...], (((1,), (1,)), ((), ())),
                         preferred_element_type=jnp.float32)
    xp_ref[...] = xp
    a_ref[...] = jnp.dot(xp, m_ref[...], preferred_element_type=jnp.float32)


def _proj(x, W_gat, M):
    return pl.pallas_call(
        _proj_body,
        grid=(_N // _BLK,),
        in_specs=[
            pl.BlockSpec((_BLK, _D), lambda i: (i, 0)),
            pl.BlockSpec((_H * _D, _D), lambda i: (0, 0)),
            pl.BlockSpec((_H * _D, 2 * _H), lambda i: (0, 0)),
        ],
        out_specs=[
            pl.BlockSpec((_BLK, _H * _D), lambda i: (i, 0)),
            pl.BlockSpec((_BLK, 2 * _H), lambda i: (i, 0)),
        ],
        out_shape=[
            jax.ShapeDtypeStruct((_N, _H * _D), jnp.float32),
            jax.ShapeDtypeStruct((_N, 2 * _H), jnp.float32),
        ],
    )(x, W_gat, M)


def _i16(v):
    return jnp.full((16,), v, dtype=jnp.int32)


def _f16(v):
    return jnp.full((16,), v, dtype=jnp.float32)


def _edge_body(src_hbm, dst_hbm, asrc_hbm, adst_hbm, xp_hbm, out_hbm,
               dbuf0, sbuf0, dbuf1, sbuf1, msrc, mdst, bsrc, bdst,
               asrc_b, adst_local, w_b, rows_b, acc, den,
               semd0, sems0, semd1, sems1, semf0, semf1, semf2):
    iota = lax.iota(jnp.int32, 16)
    wid = lax.axis_index("s") * 2 + lax.axis_index("c")

    def flush(off, nv, gbase):
        for t in range(2):
            lane = iota + (16 * t)
            g = _i16(off) + lane
            sv = plsc.load_gather(msrc, [g])
            dv = plsc.load_gather(mdst, [g])
            valid = lane < _i16(nv)
            sv = jnp.where(valid, sv, 0)
            dv = jnp.where(valid, dv, _i16(gbase))
            plsc.store_scatter(bsrc, [lane], sv)
            plsc.store_scatter(bdst, [lane], dv)
        ca = pltpu.async_copy(asrc_hbm.at[bsrc], asrc_b, semf0)
        c0 = pltpu.async_copy(xp_hbm.at[bsrc.at[pl.ds(0, 8)]],
                              rows_b.at[pl.ds(0, 8)], semf1)
        c1 = pltpu.async_copy(xp_hbm.at[bsrc.at[pl.ds(8, 8)]],
                              rows_b.at[pl.ds(8, 8)], semf2)
        ca.wait()
        for i in range(16):
            gid = iota + (16 * i)
            ridx = gid >> 3
            cidx = gid & 7
            dl = plsc.load_gather(bdst, [ridx]) - _i16(gbase)
            al = (plsc.load_gather(asrc_b, [ridx, cidx]) +
                  plsc.load_gather(adst_local, [dl, cidx]))
            al = jnp.where(al >= 0.0, al, al * 0.2)
            w_b[pl.ds(i * 16, 16)] = jnp.exp(al)

        def edge_one(k, carry):
            dsp = plsc.load_gather(bdst, [_i16(k)]) - _i16(gbase)
            for h in range(8):
                ws = plsc.load_gather(w_b, [_i16(k * 8 + h)])
                for j in range(8):
                    cc = iota + (h * 128 + j * 16)
                    rv = plsc.load_gather(rows_b, [_i16(k), cc])
                    plsc.addupdate_scatter(acc, [dsp, cc], rv * ws)
            wrow = plsc.load_gather(w_b, [_i16(k * 8) + iota])
            plsc.addupdate_scatter(den, [dsp, iota], wrow, mask=iota < 8)
            return carry

        c0.wait()
        c2 = pltpu.async_copy(xp_hbm.at[bsrc.at[pl.ds(16, 8)]],
                              rows_b.at[pl.ds(16, 8)], semf1)
        lax.fori_loop(0, jnp.minimum(nv, 8), edge_one, 0)
        c1.wait()
        c3 = pltpu.async_copy(xp_hbm.at[bsrc.at[pl.ds(24, 8)]],
                              rows_b.at[pl.ds(24, 8)], semf2)
        lax.fori_loop(8, jnp.clip(nv, 8, 16), edge_one, 0)
        c2.wait()
        lax.fori_loop(16, jnp.clip(nv, 16, 24), edge_one, 0)
        c3.wait()
        lax.fori_loop(24, jnp.clip(nv, 24, 32), edge_one, 0)

    def run_pass(p, carry):
        gbase = p * (_NW * _CT) + wid * _CT

        def zero_row(i, c):
            for j in range(64):
                cc = iota + (j * 16)
                plsc.store_scatter(acc, [_i16(i), cc], _f16(0.0))
            plsc.store_scatter(den, [_i16(i), iota], _f16(0.0),
                               mask=iota < 8)
            return c

        lax.fori_loop(0, _CT, zero_row, 0)

        cb = jnp.minimum(gbase, _N - _CT)
        pltpu.sync_copy(adst_hbm.at[pl.ds(cb, _CT)], adst_local)

        def scan_chunk(dbuf, sbuf, b, cnt_v):
            lo = _i16(gbase)
            ct = jnp.full((16,), _CT, dtype=jnp.uint32)

            def scan_vec4(v, cnt_v):
                base0 = _i16(v * 64) + iota
                for t in range(4):
                    base = base0 + (16 * t)
                    d16 = plsc.load_gather(dbuf, [base])
                    m = (d16 - lo).astype(jnp.uint32) < ct
                    mi = m.astype(jnp.int32)
                    pc = plsc.all_reduce_population_count(m)
                    pos = cnt_v + plsc.cumsum(mi) - 1
                    s16 = plsc.load_gather(sbuf, [base])
                    plsc.store_scatter(mdst, [pos], d16, mask=m)
                    plsc.store_scatter(msrc, [pos], s16, mask=m)
                    cnt_v = cnt_v + pc
                return cnt_v

            cnt_v = lax.fori_loop(0, _SCAN // 64, scan_vec4, cnt_v)
            cnt = jnp.max(cnt_v)

            def fcond(st):
                off, c = st
                return c - off >= _K

            def fbody(st):
                off, c = st
                flush(off, _K, gbase)
                return (off + _K, c)

            off, cnt = lax.while_loop(fcond, fbody, (0, cnt))

            for t in range(2):
                lane = iota + (16 * t)
                g = _i16(off) + lane
                sv = plsc.load_gather(msrc, [g])
                dv = plsc.load_gather(mdst, [g])
                plsc.store_scatter(msrc, [lane], sv)
                plsc.store_scatter(mdst, [lane], dv)
            return _i16(cnt - off)

        def issue(b, dbuf, sbuf, semd, sems):
            cd = pltpu.async_copy(dst_hbm.at[pl.ds(b * _SCAN, _SCAN)],
                                  dbuf, semd)
            cs = pltpu.async_copy(src_hbm.at[pl.ds(b * _SCAN, _SCAN)],
                                  sbuf, sems)
            return cd, cs

        issue(0, dbuf0, sbuf0, semd0, sems0)

        def block2(i, cnt_v):
            b0 = 2 * i
            issue(b0 + 1, dbuf1, sbuf1, semd1, sems1)
            pltpu.make_async_copy(dst_hbm.at[pl.ds(b0 * _SCAN, _SCAN)],
                                  dbuf0, semd0).wait()
            pltpu.make_async_copy(src_hbm.at[pl.ds(b0 * _SCAN, _SCAN)],
                                  sbuf0, sems0).wait()
            cnt_v = scan_chunk(dbuf0, sbuf0, b0, cnt_v)
            nxt = jnp.minimum(b0 + 2, _NBLK - 1)
            issue(nxt, dbuf0, sbuf0, semd0, sems0)
            pltpu.make_async_copy(dst_hbm.at[pl.ds(b0 * _SCAN, _SCAN)],
                                  dbuf1, semd1).wait()
            pltpu.make_async_copy(src_hbm.at[pl.ds(b0 * _SCAN, _SCAN)],
                                  sbuf1, sems1).wait()
            cnt_v = scan_chunk(dbuf1, sbuf1, b0 + 1, cnt_v)
            return cnt_v

        cnt_v = lax.fori_loop(0, _NBLK // 2, block2, _i16(0))
        pltpu.make_async_copy(dst_hbm.at[pl.ds(0, _SCAN)], dbuf0, semd0).wait()
        pltpu.make_async_copy(src_hbm.at[pl.ds(0, _SCAN)], sbuf0, sems0).wait()

        rem = jnp.max(cnt_v)
        flush(0, rem, gbase)

        @pl.when(gbase + _CT <= _N)
        def _():
            def nrm_row(i, c):
                si = _i16(i)
                for h in range(8):
                    dv = plsc.load_gather(den, [si, _i16(h)])
                    rs = 1.0 / dv
                    for j in range(8):
                        cc = iota + (h * 128 + j * 16)
                        v = plsc.load_gather(acc, [si, cc])
                        plsc.store_scatter(acc, [si, cc], v * rs)
                return c

            lax.fori_loop(0, _CT, nrm_row, 0)
            pltpu.sync_copy(acc, out_hbm.at[pl.ds(gbase, _CT)])

        return carry

    lax.fori_loop(0, _NPASS, run_pass, 0)


_edge_call = functools.partial(
    pl.kernel,
    _edge_body,
    out_type=jax.ShapeDtypeStruct((_N, _H * _D), jnp.float32),
    mesh=plsc.VectorSubcoreMesh(core_axis_name="c", subcore_axis_name="s"),
    compiler_params=pltpu.CompilerParams(use_tc_tiling_on_sc=False,
                                         needs_layout_passes=False),
    scratch_types=[
        pltpu.VMEM((_SCAN,), jnp.int32),
        pltpu.VMEM((_SCAN,), jnp.int32),
        pltpu.VMEM((_SCAN,), jnp.int32),
        pltpu.VMEM((_SCAN,), jnp.int32),
        pltpu.VMEM((_MCAP,), jnp.int32),
        pltpu.VMEM((_MCAP,), jnp.int32),
        pltpu.VMEM((_K,), jnp.int32),
        pltpu.VMEM((_K,), jnp.int32),
        pltpu.VMEM((_K, _H), jnp.float32),
        pltpu.VMEM((_CT, _H), jnp.float32),
        pltpu.VMEM((_K * _H,), jnp.float32),
        pltpu.VMEM((_K, _H * _D), jnp.float32),
        pltpu.VMEM((_CT, _H * _D), jnp.float32),
        pltpu.VMEM((_CT, _H), jnp.float32),
        pltpu.SemaphoreType.DMA,
        pltpu.SemaphoreType.DMA,
        pltpu.SemaphoreType.DMA,
        pltpu.SemaphoreType.DMA,
        pltpu.SemaphoreType.DMA,
        pltpu.SemaphoreType.DMA,
        pltpu.SemaphoreType.DMA,
    ],
)()


def _tail_body(agg_ref, x_ref, bgat_ref, wc_ref, bc_ref, wih_ref, whh_ref,
               bih_ref, bhh_ref, wo_ref, bo_ref, out_ref):
    m = jnp.maximum(agg_ref[...] + bgat_ref[...], 0.0)
    m = lax.dot_general(m, wc_ref[...], (((1,), (1,)), ((), ())),
                        preferred_element_type=jnp.float32)
    m = jnp.maximum(m + bc_ref[...], 0.0)
    x = x_ref[...]
    gi = lax.dot_general(m, wih_ref[...], (((1,), (1,)), ((), ())),
                         preferred_element_type=jnp.float32) + bih_ref[...]
    gh = lax.dot_general(x, whh_ref[...], (((1,), (1,)), ((), ())),
                         preferred_element_type=jnp.float32) + bhh_ref[...]
    i_r = gi[:, :_D]
    i_z = gi[:, _D:2 * _D]
    i_n = gi[:, 2 * _D:]
    h_r = gh[:, :_D]
    h_z = gh[:, _D:2 * _D]
    h_n = gh[:, 2 * _D:]
    r = jax.nn.sigmoid(i_r + h_r)
    z = jax.nn.sigmoid(i_z + h_z)
    n = jnp.tanh(i_n + r * h_n)
    h = jnp.tanh((1.0 - z) * n + z * x)
    out_ref[...] = lax.dot_general(h, wo_ref[...], (((1,), (1,)), ((), ())),
                                   preferred_element_type=jnp.float32) + bo_ref[...]


def _tail(agg, x, b_gat, W_comp, b_comp, W_ih, W_hh, b_ih, b_hh, W_opt, b_opt):
    row = lambda i: (i, 0)
    whole = lambda i: (0, 0)
    return pl.pallas_call(
        _tail_body,
        grid=(_N // _BLK,),
        in_specs=[
            pl.BlockSpec((_BLK, _H * _D), row),
            pl.BlockSpec((_BLK, _D), row),
            pl.BlockSpec((1, _H * _D), whole),
            pl.BlockSpec((_D, _H * _D), whole),
            pl.BlockSpec((1, _D), whole),
            pl.BlockSpec((3 * _D, _D), whole),
            pl.BlockSpec((3 * _D, _D), whole),
            pl.BlockSpec((1, 3 * _D), whole),
            pl.BlockSpec((1, 3 * _D), whole),
            pl.BlockSpec((_D, _D), whole),
            pl.BlockSpec((1, _D), whole),
        ],
        out_specs=pl.BlockSpec((_BLK, _D), row),
        out_shape=jax.ShapeDtypeStruct((_N, _D), jnp.float32),
    )(agg, x, b_gat.reshape(1, -1), W_comp, b_comp.reshape(1, -1),
      W_ih, W_hh, b_ih.reshape(1, -1), b_hh.reshape(1, -1),
      W_opt, b_opt.reshape(1, -1))


def kernel(x, edge_index, W_gat, att_src, att_dst, b_gat, W_comp, b_comp,
           W_ih, W_hh, b_ih, b_hh, W_opt, b_opt):
    eye = jnp.eye(_H, dtype=jnp.float32)
    m_src = (eye[:, None, :] * att_src[:, :, None]).reshape(_H * _D, _H)
    m_dst = (eye[:, None, :] * att_dst[:, :, None]).reshape(_H * _D, _H)
    M = jnp.concatenate([m_src, m_dst], axis=1)

    loop = jnp.arange(_N, dtype=edge_index.dtype)
    pad = _EP - (_E + _N)
    src = jnp.concatenate([edge_index[0], loop,
                           jnp.zeros((pad,), edge_index.dtype)])
    dst = jnp.concatenate([edge_index[1], loop,
                           jnp.full((pad,), 1 << 30, edge_index.dtype)])

    xp, a = _proj(x, W_gat, M)
    agg = _edge_call(src, dst, a[:, :_H], a[:, _H:], xp)
    return _tail(agg, x, b_gat, W_comp, b_comp, W_ih, W_hh, b_ih, b_hh,
                 W_opt, b_opt)
